# baseline (speedup 1.0000x reference)
import functools

import jax
import jax.numpy as jnp
from jax import lax
from jax.experimental import pallas as pl
from jax.experimental.pallas import tpu as pltpu
from jax.experimental.pallas import tpu_sc as plsc

B = 16384
FEAT = 128
NCLS = 5
STRIDE = 160
ACC = NCLS * STRIDE
NC, NS, L = 2, 16, 16
NW = NC * NS
RPW = B // NW
UNROLL = 4
NCHUNK = 1
MAGIC = 0x5F3759DF


@functools.partial(
    pl.kernel,
    out_type=jax.ShapeDtypeStruct((NW, NCLS, STRIDE), jnp.float32),
    mesh=plsc.VectorSubcoreMesh(core_axis_name="c", subcore_axis_name="s"),
    compiler_params=pltpu.CompilerParams(needs_layout_passes=False),
    scratch_types=[
        pltpu.VMEM((RPW * FEAT,), jnp.float32),
        pltpu.VMEM((RPW,), jnp.int32),
        pltpu.VMEM((NCLS, STRIDE), jnp.float32),
    ] + [pltpu.SemaphoreType.DMA] * NCHUNK,
)
def _sc_partials(xs_hbm, lbl_hbm, out_hbm, xs_v, lbl_v, acc_v, *sems):
    wid = lax.axis_index("s") * NC + lax.axis_index("c")
    base = pl.multiple_of(wid * RPW, RPW)
    CH = RPW // NCHUNK
    copies = [
        pltpu.async_copy(
            xs_hbm.at[pl.ds((base + c * CH) * FEAT, CH * FEAT)],
            xs_v.at[pl.ds(c * CH * FEAT, CH * FEAT)],
            sems[c],
        )
        for c in range(NCHUNK)
    ]
    pltpu.sync_copy(lbl_hbm.at[pl.ds(base, RPW)], lbl_v)

    zeros16 = jnp.zeros((L,), jnp.float32)
    for c in range(NCLS):
        for k in range(STRIDE // L):
            acc_v[c, pl.ds(k * L, L)] = zeros16

    iota = lax.iota(jnp.int32, L)
    lane0 = (iota == 0).astype(jnp.float32)
    lane1 = (iota == 1).astype(jnp.float32)
    bfly = [iota ^ st for st in (1, 2, 4, 8)]
    ioff = [iota + jnp.int32(L * j) for j in range(FEAT // L)]
    ioff_aux = iota + jnp.int32(FEAT)

    def row_work(row):
        ridx = jnp.full((L,), row, jnp.int32)
        lbl = plsc.load_gather(lbl_v, [ridx])
        xoff = pl.multiple_of(row * FEAT, FEAT)
        xj = [xs_v[pl.ds(xoff + L * j, L)] for j in range(FEAT // L)]
        sq = [x * x for x in xj]
        while len(sq) > 1:
            sq = [a + b for a, b in zip(sq[0::2], sq[1::2])]
        sv = sq[0]
        for bi in bfly:
            sv = sv + sv[bi]
        mv = jnp.maximum(sv, jnp.float32(1e-24))
        iv = jnp.int32(MAGIC) - lax.shift_right_logical(plsc.bitcast(mv, jnp.int32), 1)
        y = plsc.bitcast(iv, jnp.float32)
        half_m = mv * jnp.float32(0.5)
        for _ in range(2):
            y = y * (jnp.float32(1.5) - half_m * y * y)
        for j in range(FEAT // L):
            plsc.addupdate_scatter(acc_v, [lbl, ioff[j]], xj[j] * y)
        q_v = sv * y * y
        aux = lane0 * q_v + lane1
        plsc.addupdate_scatter(acc_v, [lbl, ioff_aux], aux)

    for c in range(NCHUNK):
        copies[c].wait()
        plsc.parallel_loop(c * CH, (c + 1) * CH, unroll=UNROLL)(row_work)
    pltpu.sync_copy(acc_v, out_hbm.at[wid])


def _tc_body(p_ref, c_ref, o_ref):
    tot = jnp.sum(p_ref[...], axis=0)
    v = tot[:, 0:FEAT]
    q = tot[:, FEAT:FEAT + 1]
    n = tot[:, FEAT + 1:FEAT + 2]
    cen = c_ref[0:NCLS, :]
    s_c = (q - 2.0 * jnp.sum(v * cen, axis=1, keepdims=True)
           + n * jnp.sum(cen * cen, axis=1, keepdims=True))
    loss = jnp.sum(s_c / (n + 1.0)) * jnp.float32(1.0 / B)
    o_ref[...] = jnp.full((1, 1), loss, jnp.float32)


_tc_finish = pl.pallas_call(
    _tc_body,
    out_shape=jax.ShapeDtypeStruct((1, 1), jnp.float32),
)


def kernel(xs, label, center):
    parts = _sc_partials(xs.reshape(B * FEAT), label.astype(jnp.int32))
    out = _tc_finish(parts, center)
    return out[0, 0]


# speedup vs baseline: 1.0152x; 1.0152x over previous
import functools

import jax
import jax.numpy as jnp
from jax import lax
from jax.experimental import pallas as pl
from jax.experimental.pallas import tpu as pltpu
from jax.experimental.pallas import tpu_sc as plsc

B = 16384
FEAT = 128
NCLS = 5
STRIDE = 160
NC, NS, L = 2, 16, 16
NW = NC * NS
SC_ROWS = 8192
RPW = SC_ROWS // NW
UNROLL = 8
MAGIC = 0x5F3759DF


@functools.partial(
    pl.kernel,
    out_type=jax.ShapeDtypeStruct((NW, NCLS, STRIDE), jnp.float32),
    mesh=plsc.VectorSubcoreMesh(core_axis_name="c", subcore_axis_name="s"),
    compiler_params=pltpu.CompilerParams(needs_layout_passes=False),
    scratch_types=[
        pltpu.VMEM((RPW * FEAT,), jnp.float32),
        pltpu.VMEM((RPW,), jnp.int32),
        pltpu.VMEM((NCLS, STRIDE), jnp.float32),
    ],
)
def _sc_partials(xs_hbm, lbl_hbm, out_hbm, xs_v, lbl_v, acc_v):
    wid = lax.axis_index("s") * NC + lax.axis_index("c")
    base = pl.multiple_of(wid * RPW, RPW)
    pltpu.sync_copy(xs_hbm.at[pl.ds(base * FEAT, RPW * FEAT)], xs_v)
    pltpu.sync_copy(lbl_hbm.at[pl.ds(base, RPW)], lbl_v)

    zeros16 = jnp.zeros((L,), jnp.float32)
    for c in range(NCLS):
        for k in range(STRIDE // L):
            acc_v[c, pl.ds(k * L, L)] = zeros16

    iota = lax.iota(jnp.int32, L)
    lane0 = (iota == 0).astype(jnp.float32)
    lane1 = (iota == 1).astype(jnp.float32)
    bfly = [iota ^ st for st in (1, 2, 4, 8)]
    ioff = [iota + jnp.int32(L * j) for j in range(FEAT // L)]
    ioff_aux = iota + jnp.int32(FEAT)

    def row_work(row):
        ridx = jnp.full((L,), row, jnp.int32)
        lbl = plsc.load_gather(lbl_v, [ridx])
        xoff = pl.multiple_of(row * FEAT, FEAT)
        xj = [xs_v[pl.ds(xoff + L * j, L)] for j in range(FEAT // L)]
        sq = [x * x for x in xj]
        while len(sq) > 1:
            sq = [a + b for a, b in zip(sq[0::2], sq[1::2])]
        sv = sq[0]
        for bi in bfly:
            sv = sv + sv[bi]
        mv = jnp.maximum(sv, jnp.float32(1e-24))
        iv = jnp.int32(MAGIC) - lax.shift_right_logical(plsc.bitcast(mv, jnp.int32), 1)
        y = plsc.bitcast(iv, jnp.float32)
        half_m = mv * jnp.float32(0.5)
        for _ in range(2):
            y = y * (jnp.float32(1.5) - half_m * y * y)
        for j in range(FEAT // L):
            plsc.addupdate_scatter(acc_v, [lbl, ioff[j]], xj[j] * y)
        q_v = sv * y * y
        aux = lane0 * q_v + lane1
        plsc.addupdate_scatter(acc_v, [lbl, ioff_aux], aux)

    plsc.parallel_loop(0, RPW, unroll=UNROLL)(row_work)
    pltpu.sync_copy(acc_v, out_hbm.at[wid])


def _tc_partial_body(x_ref, l_ref, o_ref):
    x = x_ref[...]
    s = jnp.sum(x * x, axis=1, keepdims=True)
    inv = 1.0 / jnp.maximum(jnp.sqrt(s), jnp.float32(1e-12))
    q = s * inv * inv
    onehot = (l_ref[...] == lax.broadcasted_iota(jnp.int32, (1, NCLS), 1)
              ).astype(jnp.float32)
    w = onehot * inv
    v = lax.dot_general(w, x, (((0,), (0,)), ((), ())),
                        preferred_element_type=jnp.float32)
    qn = lax.dot_general(
        onehot, jnp.concatenate([q, jnp.ones_like(q)], axis=1),
        (((0,), (0,)), ((), ())),
        preferred_element_type=jnp.float32)
    o_ref[...] = jnp.concatenate(
        [v, qn, jnp.zeros((NCLS, STRIDE - FEAT - 2), jnp.float32)], axis=1)


_tc_partial = pl.pallas_call(
    _tc_partial_body,
    out_shape=jax.ShapeDtypeStruct((NCLS, STRIDE), jnp.float32),
)


def _tc_finish_body(p_ref, t_ref, c_ref, o_ref):
    tot = jnp.sum(p_ref[...], axis=0) + t_ref[...]
    v = tot[:, 0:FEAT]
    q = tot[:, FEAT:FEAT + 1]
    n = tot[:, FEAT + 1:FEAT + 2]
    cen = c_ref[0:NCLS, :]
    s_c = (q - 2.0 * jnp.sum(v * cen, axis=1, keepdims=True)
           + n * jnp.sum(cen * cen, axis=1, keepdims=True))
    loss = jnp.sum(s_c / (n + 1.0)) * jnp.float32(1.0 / B)
    o_ref[...] = jnp.full((1, 1), loss, jnp.float32)


_tc_finish = pl.pallas_call(
    _tc_finish_body,
    out_shape=jax.ShapeDtypeStruct((1, 1), jnp.float32),
)


def kernel(xs, label, center):
    label = label.astype(jnp.int32)
    parts = _sc_partials(xs.reshape(B * FEAT), label)
    tc_part = _tc_partial(xs[SC_ROWS:], label[SC_ROWS:, None])
    out = _tc_finish(parts, tc_part, center)
    return out[0, 0]


# speedup vs baseline: 1.1065x; 1.0899x over previous
import functools

import jax
import jax.numpy as jnp
from jax import lax
from jax.experimental import pallas as pl
from jax.experimental.pallas import tpu as pltpu
from jax.experimental.pallas import tpu_sc as plsc

B = 16384
FEAT = 128
NCLS = 5
STRIDE = 160
NC, NS, L = 2, 16, 16
NW = NC * NS
SC_ROWS = 8192
RPW = SC_ROWS // NW
UNROLL = 8
MAGIC = 0x5F3759DF


@functools.partial(
    pl.kernel,
    out_type=jax.ShapeDtypeStruct((NW, NCLS, STRIDE), jnp.float32),
    mesh=plsc.VectorSubcoreMesh(core_axis_name="c", subcore_axis_name="s"),
    compiler_params=pltpu.CompilerParams(needs_layout_passes=False),
    scratch_types=[
        pltpu.VMEM((RPW * FEAT,), jnp.float32),
        pltpu.VMEM((RPW,), jnp.int32),
        pltpu.VMEM((NCLS, STRIDE), jnp.float32),
    ],
)
def _sc_partials(xs_hbm, lbl_hbm, out_hbm, xs_v, lbl_v, acc_v):
    wid = lax.axis_index("s") * NC + lax.axis_index("c")
    base = pl.multiple_of(wid * RPW, RPW)
    pltpu.sync_copy(xs_hbm.at[pl.ds(base * FEAT, RPW * FEAT)], xs_v)
    pltpu.sync_copy(lbl_hbm.at[pl.ds(base, RPW)], lbl_v)

    zeros16 = jnp.zeros((L,), jnp.float32)
    for c in range(NCLS):
        for k in range(STRIDE // L):
            acc_v[c, pl.ds(k * L, L)] = zeros16

    iota = lax.iota(jnp.int32, L)
    lane0 = (iota == 0).astype(jnp.float32)
    lane1 = (iota == 1).astype(jnp.float32)
    bfly = [iota ^ st for st in (1, 2, 4, 8)]
    ioff = [iota + jnp.int32(L * j) for j in range(FEAT // L)]
    ioff_aux = iota + jnp.int32(FEAT)

    def row_work(row):
        ridx = jnp.full((L,), row, jnp.int32)
        lbl = plsc.load_gather(lbl_v, [ridx])
        xoff = pl.multiple_of(row * FEAT, FEAT)
        xj = [xs_v[pl.ds(xoff + L * j, L)] for j in range(FEAT // L)]
        sq = [x * x for x in xj]
        while len(sq) > 1:
            sq = [a + b for a, b in zip(sq[0::2], sq[1::2])]
        sv = sq[0]
        for bi in bfly:
            sv = sv + sv[bi]
        mv = jnp.maximum(sv, jnp.float32(1e-24))
        iv = jnp.int32(MAGIC) - lax.shift_right_logical(plsc.bitcast(mv, jnp.int32), 1)
        y = plsc.bitcast(iv, jnp.float32)
        half_m = mv * jnp.float32(0.5)
        for _ in range(2):
            y = y * (jnp.float32(1.5) - half_m * y * y)
        for j in range(FEAT // L):
            plsc.addupdate_scatter(acc_v, [lbl, ioff[j]], xj[j] * y)
        q_v = sv * y * y
        aux = lane0 * q_v + lane1
        plsc.addupdate_scatter(acc_v, [lbl, ioff_aux], aux)

    plsc.parallel_loop(0, RPW, unroll=UNROLL)(row_work)
    pltpu.sync_copy(acc_v, out_hbm.at[wid])


def _tc_partial_body(x_ref, l_ref, o_ref):
    i = pl.program_id(0)

    @pl.when(i == 0)
    def _():
        o_ref[...] = jnp.zeros((NCLS, STRIDE), jnp.float32)

    x = x_ref[...]
    s = jnp.sum(x * x, axis=1, keepdims=True)
    inv = 1.0 / jnp.maximum(jnp.sqrt(s), jnp.float32(1e-12))
    q = s * inv * inv
    onehot = (l_ref[...] == lax.broadcasted_iota(jnp.int32, (1, NCLS), 1)
              ).astype(jnp.float32)
    w = onehot * inv
    v = lax.dot_general(w, x, (((0,), (0,)), ((), ())),
                        preferred_element_type=jnp.float32)
    qn = lax.dot_general(
        onehot, jnp.concatenate([q, jnp.ones_like(q)], axis=1),
        (((0,), (0,)), ((), ())),
        preferred_element_type=jnp.float32)
    contrib = jnp.concatenate(
        [v, qn, jnp.zeros((NCLS, STRIDE - FEAT - 2), jnp.float32)], axis=1)
    o_ref[...] = o_ref[...] + contrib


RB = 2048
SC_BLK = SC_ROWS // RB
_tc_partial = pl.pallas_call(
    _tc_partial_body,
    grid=((B - SC_ROWS) // RB,),
    in_specs=[
        pl.BlockSpec((RB, FEAT), lambda i: (SC_BLK + i, 0)),
        pl.BlockSpec((RB, 1), lambda i: (SC_BLK + i, 0)),
    ],
    out_specs=pl.BlockSpec((NCLS, STRIDE), lambda i: (0, 0)),
    out_shape=jax.ShapeDtypeStruct((NCLS, STRIDE), jnp.float32),
)


def _tc_finish_body(p_ref, t_ref, c_ref, o_ref):
    tot = jnp.sum(p_ref[...], axis=0) + t_ref[...]
    v = tot[:, 0:FEAT]
    q = tot[:, FEAT:FEAT + 1]
    n = tot[:, FEAT + 1:FEAT + 2]
    cen = c_ref[0:NCLS, :]
    s_c = (q - 2.0 * jnp.sum(v * cen, axis=1, keepdims=True)
           + n * jnp.sum(cen * cen, axis=1, keepdims=True))
    loss = jnp.sum(s_c / (n + 1.0)) * jnp.float32(1.0 / B)
    o_ref[...] = jnp.full((1, 1), loss, jnp.float32)


_tc_finish = pl.pallas_call(
    _tc_finish_body,
    out_shape=jax.ShapeDtypeStruct((1, 1), jnp.float32),
)


def kernel(xs, label, center):
    label = label.astype(jnp.int32)
    parts = _sc_partials(xs.reshape(B * FEAT), label)
    tc_part = _tc_partial(xs, label.reshape(B, 1))
    out = _tc_finish(parts, tc_part, center)
    return out[0, 0]
